# baseline (speedup 1.0000x reference)
import jax
import jax.numpy as jnp
from jax.experimental import pallas as pl
from jax.experimental.pallas import tpu as pltpu

E_EL = 800000
N_PTS = 8
N_NODES = 50000
DEG = 6

LANES = 128
E_PAD = 819200
R_E = E_PAD // LANES
BLK_R = 640
N_PAD = 51200
R_N = N_PAD // LANES


def _elem_partials_kernel(xi_ref, cf_ref, wel_ref, dirs_ref, props_ref, out_ref):
    i = pl.program_id(0)

    def c(k, comp):
        return cf_ref[3 * k + comp, :, :]

    pE = props_ref[0, :, :]
    pA = props_ref[1, :, :]
    pI = props_ref[2, :, :]
    L = props_ref[3, :, :]
    inv_L = 1.0 / L
    inv_L2 = inv_L * inv_L
    EA_L2 = pE * pA * inv_L2
    EI_L4 = pE * pI * inv_L2 * inv_L2

    wx = wel_ref[0, :, :]
    wz = wel_ref[2, :, :]
    cos_t = dirs_ref[0, :, :]
    sin_t = dirs_ref[2, :, :]
    w_ax = wx * cos_t + wz * sin_t
    w_tr = -wx * sin_t + wz * cos_t

    s_ax = jnp.float32(0.0)
    s_bend = jnp.float32(0.0)
    s_comp = jnp.float32(0.0)
    for p in range(N_PTS):
        x = xi_ref[0, p]
        theta = c(DEG - 1, 2)
        for k in range(DEG - 2, -1, -1):
            theta = theta * x + c(k, 2)
        dw = (DEG - 1) * c(DEG - 1, 1)
        for k in range(DEG - 3, -1, -1):
            dw = dw * x + (k + 1) * c(k + 1, 1)
        d2u = (DEG - 1) * (DEG - 2) * c(DEG - 1, 0)
        for k in range(DEG - 4, -1, -1):
            d2u = d2u * x + (k + 2) * (k + 1) * c(k + 2, 0)
        d4w = 24.0 * c(4, 1) + 120.0 * c(5, 1) * x

        res_ax = EA_L2 * d2u + w_ax
        res_bend = EI_L4 * d4w - w_tr
        compat = theta - dw * inv_L
        s_ax += jnp.sum(res_ax * res_ax)
        s_bend += jnp.sum(res_bend * res_bend)
        s_comp += jnp.sum(compat * compat)

    @pl.when(i == 0)
    def _():
        for k in range(8):
            out_ref[0, k] = 0.0

    out_ref[0, 0] += s_ax
    out_ref[0, 1] += s_bend
    out_ref[0, 2] += s_comp
    out_ref[0, 3] += jnp.sum(EA_L2)
    out_ref[0, 4] += jnp.sum(EI_L4)
    out_ref[0, 5] += jnp.sum(jnp.abs(w_ax))
    out_ref[0, 6] += jnp.sum(jnp.abs(w_tr))


def _node_reduce_kernel(nsum_ref, nsq_ref, ncnt_ref, out_ref):
    cnt_raw = ncnt_ref[...]
    inv = 1.0 / jnp.maximum(cnt_raw, 1.0)
    mask = (cnt_raw > 1.5).astype(jnp.float32)
    var_sum = jnp.float32(0.0)
    for comp in range(3):
        mean = nsum_ref[comp, :, :] * inv
        var = nsq_ref[comp, :, :] * inv - mean * mean
        var_sum += jnp.sum(var * mask)
    out_ref[0, 0] = var_sum
    out_ref[0, 1] = jnp.sum(mask)


@jax.jit
def kernel(xi, coeffs, line_load, connectivity, elem_directions,
           prop_E, prop_A, prop_I22, elem_lengths):
    xi_row = xi[0, :, 0].reshape(1, N_PTS)

    n1 = connectivity[:, 0]
    n2 = connectivity[:, 1]
    w_elem = (line_load[n1] + line_load[n2]) * 0.5

    pad = E_PAD - E_EL
    cf = jnp.pad(coeffs.reshape(E_EL, DEG * 3).T, ((0, 0), (0, pad)))
    cf = cf.reshape(DEG * 3, R_E, LANES)
    wel = jnp.pad(w_elem.T, ((0, 0), (0, pad))).reshape(3, R_E, LANES)
    dirs = jnp.pad(elem_directions.T, ((0, 0), (0, pad))).reshape(3, R_E, LANES)
    props = jnp.stack([
        jnp.pad(prop_E, (0, pad)),
        jnp.pad(prop_A, (0, pad)),
        jnp.pad(prop_I22, (0, pad)),
        jnp.pad(elem_lengths, (0, pad), constant_values=1.0),
    ]).reshape(4, R_E, LANES)

    grid_e = R_E // BLK_R
    partials = pl.pallas_call(
        _elem_partials_kernel,
        grid=(grid_e,),
        in_specs=[
            pl.BlockSpec(memory_space=pltpu.SMEM),
            pl.BlockSpec((DEG * 3, BLK_R, LANES), lambda i: (0, i, 0)),
            pl.BlockSpec((3, BLK_R, LANES), lambda i: (0, i, 0)),
            pl.BlockSpec((3, BLK_R, LANES), lambda i: (0, i, 0)),
            pl.BlockSpec((4, BLK_R, LANES), lambda i: (0, i, 0)),
        ],
        out_specs=pl.BlockSpec(memory_space=pltpu.SMEM),
        out_shape=jax.ShapeDtypeStruct((1, 8), jnp.float32),
    )(xi_row, cf, wel, dirs, props)[0]

    u_start = coeffs[:, 0, :]
    u_end = jnp.sum(coeffs, axis=1)
    node_sum = jnp.zeros((N_PAD, 3), jnp.float32).at[n1].add(u_start).at[n2].add(u_end)
    node_sq = jnp.zeros((N_PAD, 3), jnp.float32).at[n1].add(u_start * u_start).at[n2].add(u_end * u_end)
    ones = jnp.ones((E_EL,), jnp.float32)
    node_cnt = jnp.zeros((N_PAD,), jnp.float32).at[n1].add(ones).at[n2].add(ones)

    node_out = pl.pallas_call(
        _node_reduce_kernel,
        grid=(1,),
        in_specs=[
            pl.BlockSpec((3, R_N, LANES), lambda i: (0, 0, 0)),
            pl.BlockSpec((3, R_N, LANES), lambda i: (0, 0, 0)),
            pl.BlockSpec((R_N, LANES), lambda i: (0, 0)),
        ],
        out_specs=pl.BlockSpec(memory_space=pltpu.SMEM),
        out_shape=jax.ShapeDtypeStruct((1, 2), jnp.float32),
    )(node_sum.T.reshape(3, R_N, LANES),
      node_sq.T.reshape(3, R_N, LANES),
      node_cnt.reshape(R_N, LANES))[0]

    n_pts_total = jnp.float32(E_EL * N_PTS)
    inv_E = jnp.float32(1.0 / E_EL)
    EA_scale = jnp.maximum(partials[3] * inv_E, 1e-10)
    EI_scale = jnp.maximum(partials[4] * inv_E, 1e-10)
    w_ax_scale = jnp.maximum(partials[5] * inv_E, 1e-10)
    w_tr_scale = jnp.maximum(partials[6] * inv_E, 1e-10)

    loss_axial = partials[0] / n_pts_total / (EA_scale + w_ax_scale) ** 2
    loss_bending = partials[1] / n_pts_total / (EI_scale + w_tr_scale) ** 2
    loss_compat = partials[2] / n_pts_total

    var_sum = node_out[0]
    mcount = node_out[1]
    loss_cont = jnp.where(mcount > 0,
                          var_sum / jnp.maximum(3.0 * mcount, 1.0),
                          jnp.float32(0.0))

    return loss_axial + loss_bending + loss_compat + loss_cont
